# baseline (speedup 1.0000x reference)
import functools

import jax
import jax.numpy as jnp
from jax.experimental import pallas as pl
from jax.experimental.pallas import tpu as pltpu

N_TICKS = 550
N_PMTS = 12
SI = 47
S_RAW = SI * SI
BIN_SIGMA = 0.1
GAUSS_NORM = 1.0 / (BIN_SIGMA * 2.5066282746)

S_PAD = 2304
T_PAD = 640
BLK_E = 1024


def _sigmoid(x):
    return 1.0 / (1.0 + jnp.exp(-x))


def _fused_kernel(eT_ref, e_ref, w_ref,
                  wp1_ref, bp1_ref, wp2_ref, bp2_ref, ps2_ref,
                  ws1_ref, bs1_ref, ws2_ref, bs2_ref, ws3_ref, bs3_ref,
                  ws4_ref, bs4_ref, si2_ref,
                  pmt_ref, sipm_ref):
    j = pl.program_id(1)

    xyT = eT_ref[0, 0:2, :]
    zcol = e_ref[0, :, 2:3]
    wcol = w_ref[0, :, 0:1]

    h = _sigmoid(jnp.dot(ws1_ref[...], xyT,
                         preferred_element_type=jnp.float32) + bs1_ref[...])
    h = _sigmoid(jnp.dot(ws2_ref[...], h,
                         preferred_element_type=jnp.float32) + bs2_ref[...])
    h = _sigmoid(jnp.dot(ws3_ref[...], h,
                         preferred_element_type=jnp.float32) + bs3_ref[...])
    r = jnp.dot(ws4_ref[...], h.astype(jnp.bfloat16),
                preferred_element_type=jnp.float32)
    respT = (_sigmoid(r + bs4_ref[...]) * si2_ref[...]).astype(jnp.bfloat16)

    hp = _sigmoid(jnp.dot(wp1_ref[...], xyT,
                          preferred_element_type=jnp.float32) + bp1_ref[...])
    prespT = (_sigmoid(jnp.dot(wp2_ref[...], hp,
                               preferred_element_type=jnp.float32)
                       + bp2_ref[...]) * ps2_ref[...])

    ticks = (jax.lax.broadcasted_iota(jnp.int32, (BLK_E, T_PAD), 1)
             .astype(jnp.float32) + 0.5)
    d = ticks - zcol
    g = (GAUSS_NORM * wcol) * jnp.exp(d * d * (-1.0 / BIN_SIGMA))
    g_bf = g.astype(jnp.bfloat16)

    contrib_s = jnp.dot(respT, g_bf, preferred_element_type=jnp.float32)
    contrib_p = jnp.dot(prespT.astype(jnp.bfloat16), g_bf,
                        preferred_element_type=jnp.float32)

    @pl.when(j == 0)
    def _init():
        sipm_ref[0] = contrib_s
        pmt_ref[0] = contrib_p

    @pl.when(j > 0)
    def _acc():
        sipm_ref[0] += contrib_s
        pmt_ref[0] += contrib_p


@functools.partial(jax.jit, static_argnames=("interpret",))
def kernel(electrons, weight, Wp1, bp1, Wp2, bp2, pmt_scale,
           Ws1, bs1, Ws2, bs2, Ws3, bs3, Ws4, bs4, si_scale,
           interpret=False):
    B, N, _ = electrons.shape
    nblk = N // BLK_E

    eT = electrons.transpose(0, 2, 1)
    wcol = weight[..., None]

    def col(v, pad_to=None):
        if pad_to is not None:
            v = jnp.pad(v, (0, pad_to - v.shape[0]))
        return v[:, None]

    ws4T = jnp.pad(Ws4.T, ((0, S_PAD - S_RAW), (0, 0))).astype(jnp.bfloat16)
    consts = [
        Wp1.T, col(bp1), Wp2.T, col(bp2), col(pmt_scale**2),
        Ws1.T, col(bs1), Ws2.T, col(bs2), Ws3.T, col(bs3),
        ws4T, col(bs4, S_PAD), col(si_scale**2, S_PAD),
    ]

    const_specs = [
        pl.BlockSpec(c.shape, lambda b, j: (0, 0)) for c in consts
    ]
    grid_spec = pl.GridSpec(
        grid=(B, nblk),
        in_specs=[
            pl.BlockSpec((1, 3, BLK_E), lambda b, j: (b, 0, j)),
            pl.BlockSpec((1, BLK_E, 3), lambda b, j: (b, j, 0)),
            pl.BlockSpec((1, BLK_E, 1), lambda b, j: (b, j, 0)),
        ] + const_specs,
        out_specs=[
            pl.BlockSpec((1, N_PMTS, T_PAD), lambda b, j: (b, 0, 0)),
            pl.BlockSpec((1, S_PAD, T_PAD), lambda b, j: (b, 0, 0)),
        ],
    )
    pmt_out, sipm_out = pl.pallas_call(
        _fused_kernel,
        grid_spec=grid_spec,
        out_shape=[
            jax.ShapeDtypeStruct((B, N_PMTS, T_PAD), jnp.float32),
            jax.ShapeDtypeStruct((B, S_PAD, T_PAD), jnp.float32),
        ],
        compiler_params=pltpu.CompilerParams(
            dimension_semantics=("parallel", "arbitrary"),
            vmem_limit_bytes=60 * 1024 * 1024,
        ),
        interpret=interpret,
    )(eT, electrons, wcol, *consts)

    pmt_result = pmt_out[:, :, :N_TICKS]
    sipm_result = sipm_out[:, :S_RAW, :N_TICKS].reshape(B, SI, SI, N_TICKS)
    return pmt_result, sipm_result


# speedup vs baseline: 1.5019x; 1.5019x over previous
import functools

import jax
import jax.numpy as jnp
from jax.experimental import pallas as pl
from jax.experimental.pallas import tpu as pltpu

N_TICKS = 550
N_PMTS = 12
SI = 47
S_RAW = SI * SI
BIN_SIGMA = 0.1
GAUSS_NORM = 1.0 / (BIN_SIGMA * 2.5066282746)

S_PAD = 2304
T_PAD = 640
BLK_E = 1024


def _sigmoid(x):
    return 1.0 / (1.0 + jnp.exp(-x))


def _fused_kernel(eT_ref, e_ref, w_ref,
                  wp1_ref, bp1_ref, wp2_ref, bp2_ref, ps2_ref,
                  ws1_ref, bs1_ref, ws2_ref, bs2_ref, ws3_ref, bs3_ref,
                  ws4_ref, bs4_ref, si2_ref,
                  pmt_ref, sipm_ref, pacc_ref, sacc_ref, *, nblk):
    j = pl.program_id(1)

    xyT = eT_ref[0, 0:2, :]
    zcol = e_ref[0, :, 2:3]
    wcol = w_ref[0, :, 0:1]

    h = _sigmoid(jnp.dot(ws1_ref[...], xyT,
                         preferred_element_type=jnp.float32) + bs1_ref[...])
    h = _sigmoid(jnp.dot(ws2_ref[...], h,
                         preferred_element_type=jnp.float32) + bs2_ref[...])
    h = _sigmoid(jnp.dot(ws3_ref[...], h,
                         preferred_element_type=jnp.float32) + bs3_ref[...])
    r = jnp.dot(ws4_ref[...], h.astype(jnp.bfloat16),
                preferred_element_type=jnp.float32)
    respT = (_sigmoid(r + bs4_ref[...]) * si2_ref[...]).astype(jnp.bfloat16)

    hp = _sigmoid(jnp.dot(wp1_ref[...], xyT,
                          preferred_element_type=jnp.float32) + bp1_ref[...])
    prespT = (_sigmoid(jnp.dot(wp2_ref[...], hp,
                               preferred_element_type=jnp.float32)
                       + bp2_ref[...]) * ps2_ref[...])

    ticks = (jax.lax.broadcasted_iota(jnp.int32, (BLK_E, T_PAD), 1)
             .astype(jnp.float32) + 0.5)
    d = ticks - zcol
    g = (GAUSS_NORM * wcol) * jnp.exp(d * d * (-1.0 / BIN_SIGMA))
    g_bf = g.astype(jnp.bfloat16)

    contrib_s = jnp.dot(respT, g_bf, preferred_element_type=jnp.float32)
    contrib_p = jnp.dot(prespT.astype(jnp.bfloat16), g_bf,
                        preferred_element_type=jnp.float32)

    @pl.when(j == 0)
    def _init():
        sacc_ref[...] = contrib_s
        pacc_ref[...] = contrib_p

    @pl.when(j > 0)
    def _acc():
        sacc_ref[...] += contrib_s
        pacc_ref[...] += contrib_p

    @pl.when(j == nblk - 1)
    def _finish():
        sipm_ref[0] = sacc_ref[:S_RAW, :N_TICKS]
        pmt_ref[0] = pacc_ref[:N_PMTS, :N_TICKS]


@functools.partial(jax.jit, static_argnames=("interpret",))
def kernel(electrons, weight, Wp1, bp1, Wp2, bp2, pmt_scale,
           Ws1, bs1, Ws2, bs2, Ws3, bs3, Ws4, bs4, si_scale,
           interpret=False):
    B, N, _ = electrons.shape
    nblk = N // BLK_E

    eT = electrons.transpose(0, 2, 1)
    wcol = weight[..., None]

    def col(v, pad_to=None):
        if pad_to is not None:
            v = jnp.pad(v, (0, pad_to - v.shape[0]))
        return v[:, None]

    ws4T = jnp.pad(Ws4.T, ((0, S_PAD - S_RAW), (0, 0))).astype(jnp.bfloat16)
    consts = [
        Wp1.T, col(bp1), Wp2.T, col(bp2), col(pmt_scale**2),
        Ws1.T, col(bs1), Ws2.T, col(bs2), Ws3.T, col(bs3),
        ws4T, col(bs4, S_PAD), col(si_scale**2, S_PAD),
    ]

    const_specs = [
        pl.BlockSpec(c.shape, lambda b, j: (0, 0)) for c in consts
    ]
    grid_spec = dict(
        grid=(B, nblk),
        in_specs=[
            pl.BlockSpec((1, 3, BLK_E), lambda b, j: (b, 0, j)),
            pl.BlockSpec((1, BLK_E, 3), lambda b, j: (b, j, 0)),
            pl.BlockSpec((1, BLK_E, 1), lambda b, j: (b, j, 0)),
        ] + const_specs,
        out_specs=[
            pl.BlockSpec((1, N_PMTS, N_TICKS), lambda b, j: (b, 0, 0)),
            pl.BlockSpec((1, S_RAW, N_TICKS), lambda b, j: (b, 0, 0)),
        ],
    )
    pmt_out, sipm_out = pl.pallas_call(
        functools.partial(_fused_kernel, nblk=nblk),
        **grid_spec,
        out_shape=[
            jax.ShapeDtypeStruct((B, N_PMTS, N_TICKS), jnp.float32),
            jax.ShapeDtypeStruct((B, S_RAW, N_TICKS), jnp.float32),
        ],
        scratch_shapes=[
            pltpu.VMEM((N_PMTS, T_PAD), jnp.float32),
            pltpu.VMEM((S_PAD, T_PAD), jnp.float32),
        ],
        compiler_params=pltpu.CompilerParams(
            dimension_semantics=("parallel", "arbitrary"),
            vmem_limit_bytes=60 * 1024 * 1024,
        ),
        interpret=interpret,
    )(eT, electrons, wcol, *consts)

    return pmt_out, sipm_out.reshape(B, SI, SI, N_TICKS)
